# Initial kernel scaffold; baseline (speedup 1.0000x reference)
import functools

import jax
import jax.numpy as jnp
from jax import lax
from jax.experimental import pallas as pl
from jax.experimental.pallas import tpu as pltpu
from jax.experimental.pallas import tpu_sc as plsc

N = 10000
E = 320000
D = 128
DE = 16
H = 128

NC = 2
NS = 16
NW = NC * NS

EPW = E // NW
CW = 80
NCH = EPW // CW
RPT = N // NS

_mesh = plsc.VectorSubcoreMesh(core_axis_name="c", subcore_axis_name="s")


def _proj_body(x_ref, wa_ref, wb_ref, xa_ref, xb_ref):
    xv = x_ref[...]
    xa_ref[...] = jnp.dot(xv, wa_ref[...], preferred_element_type=jnp.float32)
    xb_ref[...] = jnp.dot(xv, wb_ref[...], preferred_element_type=jnp.float32)


def _proj(x, wa, wb):
    blk = 1000
    return pl.pallas_call(
        _proj_body,
        grid=(N // blk,),
        in_specs=[
            pl.BlockSpec((blk, D), lambda i: (i, 0)),
            pl.BlockSpec((D, H), lambda i: (0, 0)),
            pl.BlockSpec((D, H), lambda i: (0, 0)),
        ],
        out_specs=[
            pl.BlockSpec((blk, H), lambda i: (i, 0)),
            pl.BlockSpec((blk, H), lambda i: (i, 0)),
        ],
        out_shape=[
            jax.ShapeDtypeStruct((N, H), jnp.float32),
            jax.ShapeDtypeStruct((N, H), jnp.float32),
        ],
    )(x, wa, wb)


@functools.partial(
    pl.kernel,
    mesh=_mesh,
    out_type=jax.ShapeDtypeStruct((E, H), jnp.float32),
    scratch_types=[
        pltpu.VMEM((NCH, CW), jnp.int32),
        pltpu.VMEM((NCH, CW), jnp.int32),
        pltpu.VMEM((CW, H), jnp.float32),
        pltpu.VMEM((CW, H), jnp.float32),
        pltpu.SemaphoreType.DMA,
    ],
)
def _gather_k(xa_hbm, xb_hbm, row2_hbm, col2_hbm, g_hbm, ridx, cidx, ra, rb, sem):
    cid = lax.axis_index("c")
    sid = lax.axis_index("s")
    wid = sid * NC + cid
    pltpu.sync_copy(row2_hbm.at[pl.ds(wid * NCH, NCH)], ridx)
    pltpu.sync_copy(col2_hbm.at[pl.ds(wid * NCH, NCH)], cidx)

    def chunk(j, _):
        ca = pltpu.async_copy(xa_hbm.at[ridx.at[j]], ra, sem)
        cb = pltpu.async_copy(xb_hbm.at[cidx.at[j]], rb, sem)
        ca.wait()
        cb.wait()

        def addrow(r, __):
            for t in range(H // 16):
                sl = pl.ds(t * 16, 16)
                plsc.addupdate(ra.at[r, sl], rb[r, sl])
            return 0

        lax.fori_loop(0, CW, addrow, 0)
        pltpu.sync_copy(ra, g_hbm.at[pl.ds(wid * EPW + j * CW, CW)])
        return 0

    lax.fori_loop(0, NCH, chunk, 0)


def _edge_body(g_ref, ea_ref, wc_ref, b1_ref, w2_ref, b2_ref, ne_ref):
    h = g_ref[...] + jnp.dot(ea_ref[...], wc_ref[...],
                             preferred_element_type=jnp.float32) + b1_ref[...]
    h = jnp.maximum(h, 0.0)
    ne_ref[...] = jnp.dot(h, w2_ref[...],
                          preferred_element_type=jnp.float32) + b2_ref[...]


def _edge_mlp(g, ea, wc, b1e, w2, b2e):
    blk = 2000
    return pl.pallas_call(
        _edge_body,
        grid=(E // blk,),
        in_specs=[
            pl.BlockSpec((blk, H), lambda i: (i, 0)),
            pl.BlockSpec((blk, DE), lambda i: (i, 0)),
            pl.BlockSpec((DE, H), lambda i: (0, 0)),
            pl.BlockSpec((1, H), lambda i: (0, 0)),
            pl.BlockSpec((H, DE), lambda i: (0, 0)),
            pl.BlockSpec((1, DE), lambda i: (0, 0)),
        ],
        out_specs=pl.BlockSpec((blk, DE), lambda i: (i, 0)),
        out_shape=jax.ShapeDtypeStruct((E, DE), jnp.float32),
    )(g, ea, wc, b1e.reshape(1, H), w2, b2e.reshape(1, DE))


@functools.partial(
    pl.kernel,
    mesh=_mesh,
    out_type=jax.ShapeDtypeStruct((NC, N, DE), jnp.float32),
    scratch_types=[
        pltpu.VMEM((NCH, CW), jnp.int32),
        pltpu.VMEM((CW, DE), jnp.float32),
        pltpu.VMEM((RPT, DE), jnp.float32),
        pltpu.VMEM_SHARED((N, DE), jnp.float32),
    ],
)
def _scatter_k(ne_hbm, col2_hbm, out_hbm, cidx, upd, zbuf, agg_sh):
    cid = lax.axis_index("c")
    sid = lax.axis_index("s")
    wid = sid * NC + cid

    def zrow(i, _):
        zbuf[i, :] = jnp.zeros((DE,), jnp.float32)
        return 0

    lax.fori_loop(0, RPT, zrow, 0)
    pltpu.sync_copy(zbuf, agg_sh.at[pl.ds(sid * RPT, RPT)])
    plsc.subcore_barrier()

    pltpu.sync_copy(col2_hbm.at[pl.ds(wid * NCH, NCH)], cidx)

    def chunk(j, _):
        pltpu.sync_copy(ne_hbm.at[pl.ds(wid * EPW + j * CW, CW)], upd)
        pltpu.sync_copy(upd, agg_sh.at[cidx.at[j]], add=True)
        return 0

    lax.fori_loop(0, NCH, chunk, 0)
    plsc.subcore_barrier()

    pltpu.sync_copy(agg_sh.at[pl.ds(sid * RPT, RPT)], zbuf)
    pltpu.sync_copy(zbuf, out_hbm.at[cid, pl.ds(sid * RPT, RPT)])


def _node_body(x_ref, p0_ref, p1_ref, wx_ref, wa_ref, b_ref, out_ref):
    agg = p0_ref[...] + p1_ref[...]
    h = (jnp.dot(x_ref[...], wx_ref[...], preferred_element_type=jnp.float32)
         + jnp.dot(agg, wa_ref[...], preferred_element_type=jnp.float32)
         + b_ref[...])
    out_ref[...] = jnp.maximum(h, 0.0)


def _node_mlp(x, p0, p1, wx, wa, b1n):
    blk = 1000
    return pl.pallas_call(
        _node_body,
        grid=(N // blk,),
        in_specs=[
            pl.BlockSpec((blk, D), lambda i: (i, 0)),
            pl.BlockSpec((blk, DE), lambda i: (i, 0)),
            pl.BlockSpec((blk, DE), lambda i: (i, 0)),
            pl.BlockSpec((D, H), lambda i: (0, 0)),
            pl.BlockSpec((DE, H), lambda i: (0, 0)),
            pl.BlockSpec((1, H), lambda i: (0, 0)),
        ],
        out_specs=pl.BlockSpec((blk, H), lambda i: (i, 0)),
        out_shape=jax.ShapeDtypeStruct((N, H), jnp.float32),
    )(x, p0, p1, wx, wa, b1n.reshape(1, H))


def kernel(x, edge_index, edge_attr, W1e, b1e, W2e, b2e, W1n, b1n):
    row = edge_index[0]
    col = edge_index[1]
    row2 = row.reshape(E // CW, CW)
    col2 = col.reshape(E // CW, CW)

    wa = W1e[:D]
    wb = W1e[D:2 * D]
    wc = W1e[2 * D:]

    xa, xb = _proj(x, wa, wb)
    g = _gather_k(xa, xb, row2, col2)
    ne = _edge_mlp(g, edge_attr, wc, b1e, W2e, b2e)
    parts = _scatter_k(ne, col2)
    new_x = _node_mlp(x, parts[0], parts[1], W1n[:D], W1n[D:], b1n)
    return (new_x, ne)


# speedup vs baseline: 3.1723x; 3.1723x over previous
import functools

import jax
import jax.numpy as jnp
from jax import lax
from jax.experimental import pallas as pl
from jax.experimental.pallas import tpu as pltpu
from jax.experimental.pallas import tpu_sc as plsc

N = 10000
E = 320000
D = 128
DE = 16
H = 128

NC = 2
NS = 16
NW = NC * NS

EPW = E // NW
CW = 80
NCH = EPW // CW

NPAD = 10240
RPT = NPAD // NS

_mesh = plsc.VectorSubcoreMesh(core_axis_name="c", subcore_axis_name="s")


def _proj_body(x_ref, wa_ref, wb_ref, xa_ref, xb_ref):
    xv = x_ref[...]
    xa_ref[...] = jnp.dot(xv, wa_ref[...], preferred_element_type=jnp.float32)
    xb_ref[...] = jnp.dot(xv, wb_ref[...], preferred_element_type=jnp.float32)


def _proj(x, wa, wb):
    blk = 1000
    return pl.pallas_call(
        _proj_body,
        grid=(N // blk,),
        in_specs=[
            pl.BlockSpec((blk, D), lambda i: (i, 0)),
            pl.BlockSpec((D, H), lambda i: (0, 0)),
            pl.BlockSpec((D, H), lambda i: (0, 0)),
        ],
        out_specs=[
            pl.BlockSpec((blk, H), lambda i: (i, 0)),
            pl.BlockSpec((blk, H), lambda i: (i, 0)),
        ],
        out_shape=[
            jax.ShapeDtypeStruct((N, H), jnp.float32),
            jax.ShapeDtypeStruct((N, H), jnp.float32),
        ],
    )(x, wa, wb)


@functools.partial(
    pl.kernel,
    mesh=_mesh,
    out_type=jax.ShapeDtypeStruct((E, H), jnp.float32),
    scratch_types=[
        pltpu.VMEM((NCH, CW), jnp.int32),
        pltpu.VMEM((NCH, CW), jnp.int32),
        pltpu.VMEM((CW, H), jnp.float32),
        pltpu.VMEM((CW, H), jnp.float32),
        pltpu.SemaphoreType.DMA,
    ],
)
def _gather_k(xa_hbm, xb_hbm, row2_hbm, col2_hbm, g_hbm, ridx, cidx, ra, rb, sem):
    cid = lax.axis_index("c")
    sid = lax.axis_index("s")
    wid = sid * NC + cid
    pltpu.sync_copy(row2_hbm.at[wid], ridx)
    pltpu.sync_copy(col2_hbm.at[wid], cidx)

    def chunk(j, _):
        ca = pltpu.async_copy(xa_hbm.at[ridx.at[j]], ra, sem)
        cb = pltpu.async_copy(xb_hbm.at[cidx.at[j]], rb, sem)
        ca.wait()
        cb.wait()

        def addrow(r, __):
            for t in range(H // 16):
                sl = pl.ds(t * 16, 16)
                plsc.addupdate(ra.at[r, sl], rb[r, sl])
            return 0

        lax.fori_loop(0, CW, addrow, 0)
        pltpu.sync_copy(ra, g_hbm.at[pl.ds(wid * EPW + j * CW, CW)])
        return 0

    lax.fori_loop(0, NCH, chunk, 0)


def _edge_body(g_ref, ea_ref, wc_ref, b1_ref, w2_ref, b2_ref, ne_ref):
    h = g_ref[...] + jnp.dot(ea_ref[...], wc_ref[...],
                             preferred_element_type=jnp.float32) + b1_ref[...]
    h = jnp.maximum(h, 0.0)
    ne_ref[...] = jnp.dot(h, w2_ref[...],
                          preferred_element_type=jnp.float32) + b2_ref[...]


def _edge_mlp(g, ea, wc, b1e, w2, b2e):
    blk = 2000
    return pl.pallas_call(
        _edge_body,
        grid=(E // blk,),
        in_specs=[
            pl.BlockSpec((blk, H), lambda i: (i, 0)),
            pl.BlockSpec((blk, DE), lambda i: (i, 0)),
            pl.BlockSpec((DE, H), lambda i: (0, 0)),
            pl.BlockSpec((1, H), lambda i: (0, 0)),
            pl.BlockSpec((H, DE), lambda i: (0, 0)),
            pl.BlockSpec((1, DE), lambda i: (0, 0)),
        ],
        out_specs=pl.BlockSpec((blk, DE), lambda i: (i, 0)),
        out_shape=jax.ShapeDtypeStruct((E, DE), jnp.float32),
    )(g, ea, wc, b1e.reshape(1, H), w2, b2e.reshape(1, DE))


@functools.partial(
    pl.kernel,
    mesh=_mesh,
    out_type=jax.ShapeDtypeStruct((NC, NPAD, DE), jnp.float32),
    scratch_types=[
        pltpu.VMEM((NCH, CW), jnp.int32),
        pltpu.VMEM((CW, DE), jnp.float32),
        pltpu.VMEM((RPT, DE), jnp.float32),
        pltpu.VMEM_SHARED((NPAD, DE), jnp.float32),
    ],
    compiler_params=pltpu.CompilerParams(use_tc_tiling_on_sc=False),
)
def _scatter_k(ne_hbm, col2_hbm, out_hbm, cidx, upd, zbuf, agg_sh):
    cid = lax.axis_index("c")
    sid = lax.axis_index("s")
    wid = sid * NC + cid

    @pl.when(sid == 0)
    def _zero():
        def zrow(i, _):
            zbuf[i, :] = jnp.zeros((DE,), jnp.float32)
            return 0

        lax.fori_loop(0, RPT, zrow, 0)
        for k in range(NS):
            pltpu.sync_copy(zbuf, agg_sh.at[pl.ds(k * RPT, RPT)])

    pltpu.sync_copy(col2_hbm.at[wid], cidx)
    plsc.subcore_barrier()

    def chunk(j, _):
        pltpu.sync_copy(ne_hbm.at[pl.ds(wid * EPW + j * CW, CW)], upd)
        pltpu.sync_copy(upd, agg_sh.at[cidx.at[j]], add=True)
        return 0

    lax.fori_loop(0, NCH, chunk, 0)
    plsc.subcore_barrier()

    @pl.when(sid == 0)
    def _out():
        for k in range(NS):
            pltpu.sync_copy(agg_sh.at[pl.ds(k * RPT, RPT)], zbuf)
            pltpu.sync_copy(zbuf, out_hbm.at[cid, pl.ds(k * RPT, RPT)])


def _node_body(x_ref, p0_ref, p1_ref, wx_ref, wa_ref, b_ref, out_ref):
    agg = p0_ref[...] + p1_ref[...]
    h = (jnp.dot(x_ref[...], wx_ref[...], preferred_element_type=jnp.float32)
         + jnp.dot(agg, wa_ref[...], preferred_element_type=jnp.float32)
         + b_ref[...])
    out_ref[...] = jnp.maximum(h, 0.0)


def _node_mlp(x, p0, p1, wx, wa, b1n):
    blk = 1000
    return pl.pallas_call(
        _node_body,
        grid=(N // blk,),
        in_specs=[
            pl.BlockSpec((blk, D), lambda i: (i, 0)),
            pl.BlockSpec((blk, DE), lambda i: (i, 0)),
            pl.BlockSpec((blk, DE), lambda i: (i, 0)),
            pl.BlockSpec((D, H), lambda i: (0, 0)),
            pl.BlockSpec((DE, H), lambda i: (0, 0)),
            pl.BlockSpec((1, H), lambda i: (0, 0)),
        ],
        out_specs=pl.BlockSpec((blk, H), lambda i: (i, 0)),
        out_shape=jax.ShapeDtypeStruct((N, H), jnp.float32),
    )(x, p0, p1, wx, wa, b1n.reshape(1, H))


def kernel(x, edge_index, edge_attr, W1e, b1e, W2e, b2e, W1n, b1n):
    row = edge_index[0]
    col = edge_index[1]
    row2 = row.reshape(NW, NCH, CW)
    col2 = col.reshape(NW, NCH, CW)

    wa = W1e[:D]
    wb = W1e[D:2 * D]
    wc = W1e[2 * D:]

    xa, xb = _proj(x, wa, wb)
    g = _gather_k(xa, xb, row2, col2)
    ne = _edge_mlp(g, edge_attr, wc, b1e, W2e, b2e)
    parts = _scatter_k(ne, col2)
    new_x = _node_mlp(x, parts[0, :N], parts[1, :N], W1n[:D], W1n[D:], b1n)
    return (new_x, ne)


# speedup vs baseline: 3.7361x; 1.1777x over previous
import functools

import jax
import jax.numpy as jnp
from jax import lax
from jax.experimental import pallas as pl
from jax.experimental.pallas import tpu as pltpu
from jax.experimental.pallas import tpu_sc as plsc

N = 10000
E = 320000
D = 128
DE = 16
H = 128

NC = 2
NS = 16
NW = NC * NS

EPW = E // NW
CW = 80
NCH = EPW // CW

NPAD = 10240
RPT = NPAD // NS

_mesh = plsc.VectorSubcoreMesh(core_axis_name="c", subcore_axis_name="s")


def _proj_body(x_ref, wa_ref, wb_ref, xa_ref, xb_ref):
    xv = x_ref[...]
    xa_ref[...] = jnp.dot(xv, wa_ref[...], preferred_element_type=jnp.float32)
    xb_ref[...] = jnp.dot(xv, wb_ref[...], preferred_element_type=jnp.float32)


def _proj(x, wa, wb):
    blk = 1000
    return pl.pallas_call(
        _proj_body,
        grid=(N // blk,),
        in_specs=[
            pl.BlockSpec((blk, D), lambda i: (i, 0)),
            pl.BlockSpec((D, H), lambda i: (0, 0)),
            pl.BlockSpec((D, H), lambda i: (0, 0)),
        ],
        out_specs=[
            pl.BlockSpec((blk, H), lambda i: (i, 0)),
            pl.BlockSpec((blk, H), lambda i: (i, 0)),
        ],
        out_shape=[
            jax.ShapeDtypeStruct((N, H), jnp.float32),
            jax.ShapeDtypeStruct((N, H), jnp.float32),
        ],
    )(x, wa, wb)


@functools.partial(
    pl.kernel,
    mesh=_mesh,
    out_type=jax.ShapeDtypeStruct((E, H), jnp.float32),
    scratch_types=[
        pltpu.VMEM((NCH, CW), jnp.int32),
        pltpu.VMEM((NCH, CW), jnp.int32),
        [pltpu.VMEM((CW, H), jnp.float32)] * 2,
        [pltpu.VMEM((CW, H), jnp.float32)] * 2,
        [pltpu.VMEM((CW, H), jnp.float32)] * 2,
        [pltpu.SemaphoreType.DMA] * 2,
        [pltpu.SemaphoreType.DMA] * 2,
    ],
)
def _gather_k(xa_hbm, xb_hbm, row2_hbm, col2_hbm, g_hbm, ridx, cidx,
              ra, rb, go, semA, semO):
    cid = lax.axis_index("c")
    sid = lax.axis_index("s")
    wid = sid * NC + cid
    pltpu.sync_copy(row2_hbm.at[wid], ridx)
    pltpu.sync_copy(col2_hbm.at[wid], cidx)

    def start_gather(j, b):
        pltpu.async_copy(xa_hbm.at[ridx.at[j]], ra[b], semA[b])
        pltpu.async_copy(xb_hbm.at[cidx.at[j]], rb[b], semA[b])

    def wait_gather(j, b):
        pltpu.make_async_copy(xa_hbm.at[ridx.at[j]], ra[b], semA[b]).wait()
        pltpu.make_async_copy(xb_hbm.at[cidx.at[j]], rb[b], semA[b]).wait()

    def out_copy(j, b):
        return pltpu.make_async_copy(
            go[b], g_hbm.at[pl.ds(wid * EPW + j * CW, CW)], semO[b])

    def add_rows(b):
        def addrow(r, __):
            for t in range(H // 16):
                sl = pl.ds(t * 16, 16)
                go[b][r, sl] = ra[b][r, sl] + rb[b][r, sl]
            return 0

        lax.fori_loop(0, CW, addrow, 0)

    start_gather(0, 0)
    start_gather(1, 1)

    def step(j, b):
        wait_gather(j, b)
        add_rows(b)

        @pl.when(j + 2 < NCH)
        def _():
            start_gather(j + 2, b)

        @pl.when(j >= 2)
        def _():
            out_copy(j - 2, b).wait()

        out_copy(j, b).start()

    def outer(i, _):
        step(2 * i, 0)
        step(2 * i + 1, 1)
        return 0

    lax.fori_loop(0, (NCH - 1) // 2, outer, 0)
    step(NCH - 1, 0)
    out_copy(NCH - 1, 0).wait()
    out_copy(NCH - 2, 1).wait()


def _edge_body(g_ref, ea_ref, wc_ref, b1_ref, w2_ref, b2_ref, ne_ref):
    h = g_ref[...] + jnp.dot(ea_ref[...], wc_ref[...],
                             preferred_element_type=jnp.float32) + b1_ref[...]
    h = jnp.maximum(h, 0.0)
    ne_ref[...] = jnp.dot(h, w2_ref[...],
                          preferred_element_type=jnp.float32) + b2_ref[...]


def _edge_mlp(g, ea, wc, b1e, w2, b2e):
    blk = 2000
    return pl.pallas_call(
        _edge_body,
        grid=(E // blk,),
        in_specs=[
            pl.BlockSpec((blk, H), lambda i: (i, 0)),
            pl.BlockSpec((blk, DE), lambda i: (i, 0)),
            pl.BlockSpec((DE, H), lambda i: (0, 0)),
            pl.BlockSpec((1, H), lambda i: (0, 0)),
            pl.BlockSpec((H, DE), lambda i: (0, 0)),
            pl.BlockSpec((1, DE), lambda i: (0, 0)),
        ],
        out_specs=pl.BlockSpec((blk, DE), lambda i: (i, 0)),
        out_shape=jax.ShapeDtypeStruct((E, DE), jnp.float32),
    )(g, ea, wc, b1e.reshape(1, H), w2, b2e.reshape(1, DE))


@functools.partial(
    pl.kernel,
    mesh=_mesh,
    out_type=jax.ShapeDtypeStruct((NC, NPAD, DE), jnp.float32),
    scratch_types=[
        pltpu.VMEM((NCH, CW), jnp.int32),
        pltpu.VMEM((CW, DE), jnp.float32),
        pltpu.VMEM((RPT, DE), jnp.float32),
        pltpu.VMEM_SHARED((NPAD, DE), jnp.float32),
    ],
    compiler_params=pltpu.CompilerParams(use_tc_tiling_on_sc=False),
)
def _scatter_k(ne_hbm, col2_hbm, out_hbm, cidx, upd, zbuf, agg_sh):
    cid = lax.axis_index("c")
    sid = lax.axis_index("s")
    wid = sid * NC + cid

    @pl.when(sid == 0)
    def _zero():
        def zrow(i, _):
            zbuf[i, :] = jnp.zeros((DE,), jnp.float32)
            return 0

        lax.fori_loop(0, RPT, zrow, 0)
        for k in range(NS):
            pltpu.sync_copy(zbuf, agg_sh.at[pl.ds(k * RPT, RPT)])

    pltpu.sync_copy(col2_hbm.at[wid], cidx)
    plsc.subcore_barrier()

    def chunk(j, _):
        pltpu.sync_copy(ne_hbm.at[pl.ds(wid * EPW + j * CW, CW)], upd)
        pltpu.sync_copy(upd, agg_sh.at[cidx.at[j]], add=True)
        return 0

    lax.fori_loop(0, NCH, chunk, 0)
    plsc.subcore_barrier()

    @pl.when(sid == 0)
    def _out():
        for k in range(NS):
            pltpu.sync_copy(agg_sh.at[pl.ds(k * RPT, RPT)], zbuf)
            pltpu.sync_copy(zbuf, out_hbm.at[cid, pl.ds(k * RPT, RPT)])


def _node_body(x_ref, p0_ref, p1_ref, wx_ref, wa_ref, b_ref, out_ref):
    agg = p0_ref[...] + p1_ref[...]
    h = (jnp.dot(x_ref[...], wx_ref[...], preferred_element_type=jnp.float32)
         + jnp.dot(agg, wa_ref[...], preferred_element_type=jnp.float32)
         + b_ref[...])
    out_ref[...] = jnp.maximum(h, 0.0)


def _node_mlp(x, p0, p1, wx, wa, b1n):
    blk = 1000
    return pl.pallas_call(
        _node_body,
        grid=(N // blk,),
        in_specs=[
            pl.BlockSpec((blk, D), lambda i: (i, 0)),
            pl.BlockSpec((blk, DE), lambda i: (i, 0)),
            pl.BlockSpec((blk, DE), lambda i: (i, 0)),
            pl.BlockSpec((D, H), lambda i: (0, 0)),
            pl.BlockSpec((DE, H), lambda i: (0, 0)),
            pl.BlockSpec((1, H), lambda i: (0, 0)),
        ],
        out_specs=pl.BlockSpec((blk, H), lambda i: (i, 0)),
        out_shape=jax.ShapeDtypeStruct((N, H), jnp.float32),
    )(x, p0, p1, wx, wa, b1n.reshape(1, H))


def kernel(x, edge_index, edge_attr, W1e, b1e, W2e, b2e, W1n, b1n):
    row = edge_index[0]
    col = edge_index[1]
    row2 = row.reshape(NW, NCH, CW)
    col2 = col.reshape(NW, NCH, CW)

    wa = W1e[:D]
    wb = W1e[D:2 * D]
    wc = W1e[2 * D:]

    xa, xb = _proj(x, wa, wb)
    g = _gather_k(xa, xb, row2, col2)
    ne = _edge_mlp(g, edge_attr, wc, b1e, W2e, b2e)
    parts = _scatter_k(ne, col2)
    new_x = _node_mlp(x, parts[0, :N], parts[1, :N], W1n[:D], W1n[D:], b1n)
    return (new_x, ne)


# speedup vs baseline: 3.9705x; 1.0627x over previous
import functools

import jax
import jax.numpy as jnp
from jax import lax
from jax.experimental import pallas as pl
from jax.experimental.pallas import tpu as pltpu
from jax.experimental.pallas import tpu_sc as plsc

N = 10000
E = 320000
D = 128
DE = 16
H = 128

NC = 2
NS = 16
NW = NC * NS

EPW = E // NW
CW = 80
NCH = EPW // CW

NPAD = 10240
RPT = NPAD // NS

_mesh = plsc.VectorSubcoreMesh(core_axis_name="c", subcore_axis_name="s")


def _proj_body(x_ref, wa_ref, wb_ref, xa_ref, xb_ref):
    xv = x_ref[...]
    xa_ref[...] = jnp.dot(xv, wa_ref[...], preferred_element_type=jnp.float32)
    xb_ref[...] = jnp.dot(xv, wb_ref[...], preferred_element_type=jnp.float32)


def _proj(x, wa, wb):
    blk = 1000
    return pl.pallas_call(
        _proj_body,
        grid=(N // blk,),
        in_specs=[
            pl.BlockSpec((blk, D), lambda i: (i, 0)),
            pl.BlockSpec((D, H), lambda i: (0, 0)),
            pl.BlockSpec((D, H), lambda i: (0, 0)),
        ],
        out_specs=[
            pl.BlockSpec((blk, H), lambda i: (i, 0)),
            pl.BlockSpec((blk, H), lambda i: (i, 0)),
        ],
        out_shape=[
            jax.ShapeDtypeStruct((N, H), jnp.float32),
            jax.ShapeDtypeStruct((N, H), jnp.float32),
        ],
    )(x, wa, wb)


@functools.partial(
    pl.kernel,
    mesh=_mesh,
    out_type=jax.ShapeDtypeStruct((E, H), jnp.float32),
    scratch_types=[
        pltpu.VMEM((NCH, CW), jnp.int32),
        pltpu.VMEM((NCH, CW), jnp.int32),
        [pltpu.VMEM((CW, H), jnp.float32)] * 2,
        [pltpu.VMEM((CW, H), jnp.float32)] * 2,
        [pltpu.VMEM((CW, H), jnp.float32)] * 2,
        [pltpu.SemaphoreType.DMA] * 2,
        [pltpu.SemaphoreType.DMA] * 2,
    ],
)
def _gather_k(xa_hbm, xb_hbm, row2_hbm, col2_hbm, g_hbm, ridx, cidx,
              ra, rb, go, semA, semO):
    cid = lax.axis_index("c")
    sid = lax.axis_index("s")
    wid = sid * NC + cid
    pltpu.sync_copy(row2_hbm.at[wid], ridx)
    pltpu.sync_copy(col2_hbm.at[wid], cidx)

    def start_gather(j, b):
        pltpu.async_copy(xa_hbm.at[ridx.at[j]], ra[b], semA[b])
        pltpu.async_copy(xb_hbm.at[cidx.at[j]], rb[b], semA[b])

    def wait_gather(j, b):
        pltpu.make_async_copy(xa_hbm.at[ridx.at[j]], ra[b], semA[b]).wait()
        pltpu.make_async_copy(xb_hbm.at[cidx.at[j]], rb[b], semA[b]).wait()

    def out_copy(j, b):
        return pltpu.make_async_copy(
            go[b], g_hbm.at[pl.ds(wid * EPW + j * CW, CW)], semO[b])

    def add_rows(b):
        def addrow(r, __):
            for t in range(H // 16):
                sl = pl.ds(t * 16, 16)
                go[b][r, sl] = ra[b][r, sl] + rb[b][r, sl]
            return 0

        lax.fori_loop(0, CW, addrow, 0)

    start_gather(0, 0)
    start_gather(1, 1)

    def step(j, b):
        wait_gather(j, b)
        add_rows(b)

        @pl.when(j + 2 < NCH)
        def _():
            start_gather(j + 2, b)

        @pl.when(j >= 2)
        def _():
            out_copy(j - 2, b).wait()

        out_copy(j, b).start()

    def outer(i, _):
        step(2 * i, 0)
        step(2 * i + 1, 1)
        return 0

    lax.fori_loop(0, (NCH - 1) // 2, outer, 0)
    step(NCH - 1, 0)
    out_copy(NCH - 1, 0).wait()
    out_copy(NCH - 2, 1).wait()


def _edge_body(g_ref, ea_ref, wc_ref, b1_ref, w2_ref, b2_ref, ne_ref):
    h = g_ref[...] + jnp.dot(ea_ref[...], wc_ref[...],
                             preferred_element_type=jnp.float32) + b1_ref[...]
    h = jnp.maximum(h, 0.0)
    ne_ref[...] = jnp.dot(h, w2_ref[...],
                          preferred_element_type=jnp.float32) + b2_ref[...]


def _edge_mlp(g, ea, wc, b1e, w2, b2e):
    blk = 2000
    return pl.pallas_call(
        _edge_body,
        grid=(E // blk,),
        in_specs=[
            pl.BlockSpec((blk, H), lambda i: (i, 0)),
            pl.BlockSpec((blk, DE), lambda i: (i, 0)),
            pl.BlockSpec((DE, H), lambda i: (0, 0)),
            pl.BlockSpec((1, H), lambda i: (0, 0)),
            pl.BlockSpec((H, DE), lambda i: (0, 0)),
            pl.BlockSpec((1, DE), lambda i: (0, 0)),
        ],
        out_specs=pl.BlockSpec((blk, DE), lambda i: (i, 0)),
        out_shape=jax.ShapeDtypeStruct((E, DE), jnp.float32),
    )(g, ea, wc, b1e.reshape(1, H), w2, b2e.reshape(1, DE))


@functools.partial(
    pl.kernel,
    mesh=_mesh,
    out_type=jax.ShapeDtypeStruct((NC, NPAD, DE), jnp.float32),
    scratch_types=[
        pltpu.VMEM((NCH, CW), jnp.int32),
        [pltpu.VMEM((CW, DE), jnp.float32)] * 4,
        pltpu.VMEM((RPT, DE), jnp.float32),
        pltpu.VMEM_SHARED((NPAD, DE), jnp.float32),
        [pltpu.SemaphoreType.DMA] * 4,
        [pltpu.SemaphoreType.DMA] * 4,
    ],
    compiler_params=pltpu.CompilerParams(use_tc_tiling_on_sc=False),
)
def _scatter_k(ne_hbm, col2_hbm, out_hbm, cidx, upd, zbuf, agg_sh, semI, semS):
    cid = lax.axis_index("c")
    sid = lax.axis_index("s")
    wid = sid * NC + cid

    @pl.when(sid == 0)
    def _zero():
        def zrow(i, _):
            zbuf[i, :] = jnp.zeros((DE,), jnp.float32)
            return 0

        lax.fori_loop(0, RPT, zrow, 0)
        for k in range(NS):
            pltpu.sync_copy(zbuf, agg_sh.at[pl.ds(k * RPT, RPT)])

    pltpu.sync_copy(col2_hbm.at[wid], cidx)
    plsc.subcore_barrier()

    def in_copy(j, b):
        return pltpu.make_async_copy(
            ne_hbm.at[pl.ds(wid * EPW + j * CW, CW)], upd[b], semI[b])

    def sadd(j, b):
        return pltpu.make_async_copy(upd[b], agg_sh.at[cidx.at[j]], semS[b])

    for b in range(4):
        in_copy(b, b).start()

    def step(j, b):
        in_copy(j, b).wait()

        @pl.when(j >= 4)
        def _():
            sadd(j - 4, b).wait()

        pltpu.async_copy(upd[b], agg_sh.at[cidx.at[j]], semS[b], add=True)

        @pl.when(j + 4 < NCH)
        def _():
            in_copy(j + 4, b).start()

    def outer(i, _):
        for b in range(4):
            step(4 * i + b, b)
        return 0

    lax.fori_loop(0, NCH // 4, outer, 0)
    step(NCH - 1, 0)
    sadd(NCH - 4, 1).wait()
    sadd(NCH - 3, 2).wait()
    sadd(NCH - 2, 3).wait()
    sadd(NCH - 1, 0).wait()
    plsc.subcore_barrier()

    @pl.when(sid == 0)
    def _out():
        for k in range(NS):
            pltpu.sync_copy(agg_sh.at[pl.ds(k * RPT, RPT)], zbuf)
            pltpu.sync_copy(zbuf, out_hbm.at[cid, pl.ds(k * RPT, RPT)])


def _node_body(x_ref, p0_ref, p1_ref, wx_ref, wa_ref, b_ref, out_ref):
    agg = p0_ref[...] + p1_ref[...]
    h = (jnp.dot(x_ref[...], wx_ref[...], preferred_element_type=jnp.float32)
         + jnp.dot(agg, wa_ref[...], preferred_element_type=jnp.float32)
         + b_ref[...])
    out_ref[...] = jnp.maximum(h, 0.0)


def _node_mlp(x, p0, p1, wx, wa, b1n):
    blk = 1000
    return pl.pallas_call(
        _node_body,
        grid=(N // blk,),
        in_specs=[
            pl.BlockSpec((blk, D), lambda i: (i, 0)),
            pl.BlockSpec((blk, DE), lambda i: (i, 0)),
            pl.BlockSpec((blk, DE), lambda i: (i, 0)),
            pl.BlockSpec((D, H), lambda i: (0, 0)),
            pl.BlockSpec((DE, H), lambda i: (0, 0)),
            pl.BlockSpec((1, H), lambda i: (0, 0)),
        ],
        out_specs=pl.BlockSpec((blk, H), lambda i: (i, 0)),
        out_shape=jax.ShapeDtypeStruct((N, H), jnp.float32),
    )(x, p0, p1, wx, wa, b1n.reshape(1, H))


def kernel(x, edge_index, edge_attr, W1e, b1e, W2e, b2e, W1n, b1n):
    row = edge_index[0]
    col = edge_index[1]
    row2 = row.reshape(NW, NCH, CW)
    col2 = col.reshape(NW, NCH, CW)

    wa = W1e[:D]
    wb = W1e[D:2 * D]
    wc = W1e[2 * D:]

    xa, xb = _proj(x, wa, wb)
    g = _gather_k(xa, xb, row2, col2)
    ne = _edge_mlp(g, edge_attr, wc, b1e, W2e, b2e)
    parts = _scatter_k(ne, col2)
    new_x = _node_mlp(x, parts[0, :N], parts[1, :N], W1n[:D], W1n[D:], b1n)
    return (new_x, ne)


# speedup vs baseline: 3.9747x; 1.0011x over previous
import functools

import jax
import jax.numpy as jnp
from jax import lax
from jax.experimental import pallas as pl
from jax.experimental.pallas import tpu as pltpu
from jax.experimental.pallas import tpu_sc as plsc

N = 10000
E = 320000
D = 128
DE = 16
H = 128

NC = 2
NS = 16
NW = NC * NS

EPW = E // NW
CW = 80
NCH = EPW // CW

NPAD = 10240
RPT = NPAD // NS

_mesh = plsc.VectorSubcoreMesh(core_axis_name="c", subcore_axis_name="s")


def _proj_body(x_ref, wa_ref, wb_ref, xa_ref, xb_ref):
    xv = x_ref[...]
    xa_ref[...] = jnp.dot(xv, wa_ref[...], preferred_element_type=jnp.float32)
    xb_ref[...] = jnp.dot(xv, wb_ref[...], preferred_element_type=jnp.float32)


def _proj(x, wa, wb):
    blk = 1000
    return pl.pallas_call(
        _proj_body,
        grid=(N // blk,),
        in_specs=[
            pl.BlockSpec((blk, D), lambda i: (i, 0)),
            pl.BlockSpec((D, H), lambda i: (0, 0)),
            pl.BlockSpec((D, H), lambda i: (0, 0)),
        ],
        out_specs=[
            pl.BlockSpec((blk, H), lambda i: (i, 0)),
            pl.BlockSpec((blk, H), lambda i: (i, 0)),
        ],
        out_shape=[
            jax.ShapeDtypeStruct((N, H), jnp.float32),
            jax.ShapeDtypeStruct((N, H), jnp.float32),
        ],
    )(x, wa, wb)


@functools.partial(
    pl.kernel,
    mesh=_mesh,
    out_type=jax.ShapeDtypeStruct((E, H), jnp.float32),
    scratch_types=[
        pltpu.VMEM((NCH, CW), jnp.int32),
        pltpu.VMEM((NCH, CW), jnp.int32),
        [pltpu.VMEM((CW, H), jnp.float32)] * 2,
        [pltpu.VMEM((CW, H), jnp.float32)] * 2,
        [pltpu.VMEM((CW, H), jnp.float32)] * 2,
        [pltpu.SemaphoreType.DMA] * 2,
        [pltpu.SemaphoreType.DMA] * 2,
    ],
)
def _gather_k(xa_hbm, xb_hbm, row2_hbm, col2_hbm, g_hbm, ridx, cidx,
              ra, rb, go, semA, semO):
    cid = lax.axis_index("c")
    sid = lax.axis_index("s")
    wid = sid * NC + cid
    pltpu.sync_copy(row2_hbm.at[wid], ridx)
    pltpu.sync_copy(col2_hbm.at[wid], cidx)

    def start_gather(j, b):
        pltpu.async_copy(xa_hbm.at[ridx.at[j]], ra[b], semA[b])
        pltpu.async_copy(xb_hbm.at[cidx.at[j]], rb[b], semA[b])

    def wait_gather(j, b):
        pltpu.make_async_copy(xa_hbm.at[ridx.at[j]], ra[b], semA[b]).wait()
        pltpu.make_async_copy(xb_hbm.at[cidx.at[j]], rb[b], semA[b]).wait()

    def out_copy(j, b):
        return pltpu.make_async_copy(
            go[b], g_hbm.at[pl.ds(wid * EPW + j * CW, CW)], semO[b])

    def add_rows(b):
        def addrow(r, __):
            for t in range(H // 16):
                sl = pl.ds(t * 16, 16)
                go[b][r, sl] = ra[b][r, sl] + rb[b][r, sl]
            return 0

        lax.fori_loop(0, CW, addrow, 0)

    start_gather(0, 0)
    start_gather(1, 1)

    def step(j, b):
        wait_gather(j, b)

        @pl.when(j >= 2)
        def _():
            out_copy(j - 2, b).wait()

        add_rows(b)

        @pl.when(j + 2 < NCH)
        def _():
            start_gather(j + 2, b)

        out_copy(j, b).start()

    def outer(i, _):
        step(2 * i, 0)
        step(2 * i + 1, 1)
        return 0

    lax.fori_loop(0, (NCH - 1) // 2, outer, 0)
    step(NCH - 1, 0)
    out_copy(NCH - 1, 0).wait()
    out_copy(NCH - 2, 1).wait()


def _edge_body(g_ref, ea_ref, wc_ref, b1_ref, w2_ref, b2_ref, ne_ref):
    h = g_ref[...] + jnp.dot(ea_ref[...], wc_ref[...],
                             preferred_element_type=jnp.float32) + b1_ref[...]
    h = jnp.maximum(h, 0.0)
    ne_ref[...] = jnp.dot(h, w2_ref[...],
                          preferred_element_type=jnp.float32) + b2_ref[...]


def _edge_mlp(g, ea, wc, b1e, w2, b2e):
    blk = 2000
    return pl.pallas_call(
        _edge_body,
        grid=(E // blk,),
        in_specs=[
            pl.BlockSpec((blk, H), lambda i: (i, 0)),
            pl.BlockSpec((blk, DE), lambda i: (i, 0)),
            pl.BlockSpec((DE, H), lambda i: (0, 0)),
            pl.BlockSpec((1, H), lambda i: (0, 0)),
            pl.BlockSpec((H, DE), lambda i: (0, 0)),
            pl.BlockSpec((1, DE), lambda i: (0, 0)),
        ],
        out_specs=pl.BlockSpec((blk, DE), lambda i: (i, 0)),
        out_shape=jax.ShapeDtypeStruct((E, DE), jnp.float32),
    )(g, ea, wc, b1e.reshape(1, H), w2, b2e.reshape(1, DE))


@functools.partial(
    pl.kernel,
    mesh=_mesh,
    out_type=jax.ShapeDtypeStruct((NC, NPAD, DE), jnp.float32),
    scratch_types=[
        pltpu.VMEM((NCH, CW), jnp.int32),
        [pltpu.VMEM((CW, DE), jnp.float32)] * 4,
        pltpu.VMEM((RPT, DE), jnp.float32),
        pltpu.VMEM_SHARED((NPAD, DE), jnp.float32),
        [pltpu.SemaphoreType.DMA] * 4,
        [pltpu.SemaphoreType.DMA] * 4,
    ],
    compiler_params=pltpu.CompilerParams(use_tc_tiling_on_sc=False),
)
def _scatter_k(ne_hbm, col2_hbm, out_hbm, cidx, upd, zbuf, agg_sh, semI, semS):
    cid = lax.axis_index("c")
    sid = lax.axis_index("s")
    wid = sid * NC + cid

    @pl.when(sid == 0)
    def _zero():
        def zrow(i, _):
            zbuf[i, :] = jnp.zeros((DE,), jnp.float32)
            return 0

        lax.fori_loop(0, RPT, zrow, 0)
        for k in range(NS):
            pltpu.sync_copy(zbuf, agg_sh.at[pl.ds(k * RPT, RPT)])

    pltpu.sync_copy(col2_hbm.at[wid], cidx)
    plsc.subcore_barrier()

    def in_copy(j, b):
        return pltpu.make_async_copy(
            ne_hbm.at[pl.ds(wid * EPW + j * CW, CW)], upd[b], semI[b])

    def sadd(j, b):
        return pltpu.make_async_copy(upd[b], agg_sh.at[cidx.at[j]], semS[b])

    for b in range(3):
        in_copy(b, b).start()

    def step(j, b):
        in_copy(j, b).wait()

        bp = (b - 1) % 4

        @pl.when(j >= 1)
        def _():
            sadd(j - 1, bp).wait()

        pltpu.async_copy(upd[b], agg_sh.at[cidx.at[j]], semS[b], add=True)

        @pl.when(j + 3 < NCH)
        def _():
            in_copy(j + 3, bp).start()

    def outer(i, _):
        for b in range(4):
            step(4 * i + b, b)
        return 0

    lax.fori_loop(0, NCH // 4, outer, 0)
    step(NCH - 1, 0)
    sadd(NCH - 1, 0).wait()
    plsc.subcore_barrier()

    @pl.when(sid == 0)
    def _out():
        for k in range(NS):
            pltpu.sync_copy(agg_sh.at[pl.ds(k * RPT, RPT)], zbuf)
            pltpu.sync_copy(zbuf, out_hbm.at[cid, pl.ds(k * RPT, RPT)])


def _node_body(x_ref, p0_ref, p1_ref, wx_ref, wa_ref, b_ref, out_ref):
    agg = p0_ref[...] + p1_ref[...]
    h = (jnp.dot(x_ref[...], wx_ref[...], preferred_element_type=jnp.float32)
         + jnp.dot(agg, wa_ref[...], preferred_element_type=jnp.float32)
         + b_ref[...])
    out_ref[...] = jnp.maximum(h, 0.0)


def _node_mlp(x, p0, p1, wx, wa, b1n):
    blk = 1000
    return pl.pallas_call(
        _node_body,
        grid=(N // blk,),
        in_specs=[
            pl.BlockSpec((blk, D), lambda i: (i, 0)),
            pl.BlockSpec((blk, DE), lambda i: (i, 0)),
            pl.BlockSpec((blk, DE), lambda i: (i, 0)),
            pl.BlockSpec((D, H), lambda i: (0, 0)),
            pl.BlockSpec((DE, H), lambda i: (0, 0)),
            pl.BlockSpec((1, H), lambda i: (0, 0)),
        ],
        out_specs=pl.BlockSpec((blk, H), lambda i: (i, 0)),
        out_shape=jax.ShapeDtypeStruct((N, H), jnp.float32),
    )(x, p0, p1, wx, wa, b1n.reshape(1, H))


def kernel(x, edge_index, edge_attr, W1e, b1e, W2e, b2e, W1n, b1n):
    row = edge_index[0]
    col = edge_index[1]
    row2 = row.reshape(NW, NCH, CW)
    col2 = col.reshape(NW, NCH, CW)

    wa = W1e[:D]
    wb = W1e[D:2 * D]
    wc = W1e[2 * D:]

    xa, xb = _proj(x, wa, wb)
    g = _gather_k(xa, xb, row2, col2)
    ne = _edge_mlp(g, edge_attr, wc, b1e, W2e, b2e)
    parts = _scatter_k(ne, col2)
    new_x = _node_mlp(x, parts[0, :N], parts[1, :N], W1n[:D], W1n[D:], b1n)
    return (new_x, ne)
